# baseline (speedup 1.0000x reference)
import functools

import numpy as np
import jax
import jax.numpy as jnp
from jax import lax
from jax.experimental import pallas as pl
from jax.experimental.pallas import tpu as pltpu
from jax.experimental.pallas import tpu_sc as plsc

N_NODES = 10000
N_EDGES = 160000
NP = 10240
DH = 128
EA = 160768
C = 16
C8 = C * 8
EW = C * 8 * 16 + C * 8 + C
LAYER_DIMS = [(50, 75), (75, 100), (100, 200), (200, 200), (200, 200),
              (200, 200), (200, 200), (200, 200), (200, 200), (200, 100),
              (100, 75), (75, 50)]


def _basis_kernel(ea_ref, src_ref, g56_ref, g28_ref, wb_ref):
    p = jnp.clip(ea_ref[...], 0.0, 1.0) * 2.0
    lo = jnp.minimum(jnp.floor(p), 1.0)
    frac = p - lo
    lo_i = lo.astype(jnp.int32)
    strides = (1, 3, 9)
    for s in range(8):
        b = jnp.ones((1, EA), jnp.float32)
        wi = jnp.zeros((1, EA), jnp.int32)
        for dd in range(3):
            bit = (s >> dd) & 1
            f = frac[dd:dd + 1, :]
            b = b * (f if bit else (1.0 - f))
            wi = wi + (lo_i[dd:dd + 1, :] + bit) * strides[dd]
        g56_ref[s:s + 1, :] = src_ref[...] * 56 + wi
        g28_ref[s:s + 1, :] = src_ref[...] * 28 + wi
        wb_ref[s:s + 1, :] = b


_basis = pl.pallas_call(
    _basis_kernel,
    out_shape=(jax.ShapeDtypeStruct((8, EA), jnp.int32),
               jax.ShapeDtypeStruct((8, EA), jnp.int32),
               jax.ShapeDtypeStruct((8, EA), jnp.float32)),
)


@functools.cache
def _make_mm(din_p, ns):
    cols = ns * DH
    bm = 320

    def mm_body(h_ref, w_ref, o_ref, r_ref):
        acc = jnp.dot(h_ref[...], w_ref[...], preferred_element_type=jnp.float32)
        o_ref[...] = acc
        if ns == 56:
            r_ref[...] = jnp.concatenate(
                [acc[:, 27 * DH:28 * DH], acc[:, 55 * DH:56 * DH]], axis=1)
        else:
            r_ref[...] = acc[:, 27 * DH:28 * DH]

    rw = 2 * DH if ns == 56 else DH
    return pl.pallas_call(
        mm_body,
        grid=(NP // bm,),
        in_specs=[pl.BlockSpec((bm, din_p), lambda i: (i, 0)),
                  pl.BlockSpec((din_p, cols), lambda i: (0, 0))],
        out_specs=(pl.BlockSpec((bm, cols), lambda i: (i, 0)),
                   pl.BlockSpec((bm, rw), lambda i: (i, 0))),
        out_shape=(jax.ShapeDtypeStruct((NP, cols), jnp.float32),
                   jax.ShapeDtypeStruct((NP, rw), jnp.float32)),
    )


@functools.cache
def _make_post_big(final):
    bp = 640
    gi = NP // bp

    def post_body(agg_ref, r_ref, bias_ref, g_ref, b_ref, o_ref):
        v = agg_ref[...] + r_ref[...] + bias_ref[...]
        if not final:
            v = jnp.where(v > 0, v, jnp.exp(v) - 1.0)
            v = v * g_ref[...] + b_ref[...]
        o_ref[...] = v

    return pl.pallas_call(
        post_body,
        grid=(gi, 2),
        in_specs=[pl.BlockSpec((bp, DH), lambda i, j: (j * gi + i, 0)),
                  pl.BlockSpec((bp, DH), lambda i, j: (i, j)),
                  pl.BlockSpec((1, DH), lambda i, j: (0, j)),
                  pl.BlockSpec((1, DH), lambda i, j: (0, j)),
                  pl.BlockSpec((1, DH), lambda i, j: (0, j))],
        out_specs=pl.BlockSpec((bp, DH), lambda i, j: (i, j)),
        out_shape=jax.ShapeDtypeStruct((NP, 2 * DH), jnp.float32),
    )


@functools.cache
def _make_post_small(final):
    bp = 640
    gi = NP // bp

    def post_body(a0_ref, a1_ref, r_ref, bias_ref, g_ref, b_ref, o_ref):
        v = a0_ref[...] + a1_ref[...] + r_ref[...] + bias_ref[...]
        if not final:
            v = jnp.where(v > 0, v, jnp.exp(v) - 1.0)
            v = v * g_ref[...] + b_ref[...]
        o_ref[...] = v

    return pl.pallas_call(
        post_body,
        grid=(gi,),
        in_specs=[pl.BlockSpec((bp, DH), lambda i: (i, 0)),
                  pl.BlockSpec((bp, DH), lambda i: (gi + i, 0)),
                  pl.BlockSpec((bp, DH), lambda i: (i, 0)),
                  pl.BlockSpec((1, DH), lambda i: (0, 0)),
                  pl.BlockSpec((1, DH), lambda i: (0, 0)),
                  pl.BlockSpec((1, DH), lambda i: (0, 0))],
        out_specs=pl.BlockSpec((bp, DH), lambda i: (i, 0)),
        out_shape=jax.ShapeDtypeStruct((NP, DH), jnp.float32),
    )


@functools.cache
def _make_sc(feat_split):
    ept = EA // 16 if feat_split else EA // 32
    nchunks = ept // C
    nsl = NP // 16
    mesh = plsc.VectorSubcoreMesh(core_axis_name="c", subcore_axis_name="s")

    @functools.partial(
        pl.kernel,
        out_type=jax.ShapeDtypeStruct((2 * NP, DH), jnp.float32),
        mesh=mesh,
        scratch_types=[
            pltpu.VMEM((C8 * 16,), jnp.float32),
            pltpu.VMEM((C8 * 16,), jnp.float32),
            pltpu.VMEM((C8,), jnp.int32),
            pltpu.VMEM((C8,), jnp.int32),
            pltpu.VMEM((C,), jnp.int32),
            pltpu.VMEM((C,), jnp.int32),
            pltpu.VMEM((C8,), jnp.int32),
            pltpu.VMEM((C8,), jnp.int32),
            pltpu.VMEM((C8, DH), jnp.float32),
            pltpu.VMEM((C8, DH), jnp.float32),
            pltpu.VMEM((C, DH), jnp.float32),
            pltpu.VMEM((C, DH), jnp.float32),
            pltpu.VMEM((C,), jnp.int32),
            pltpu.VMEM((C,), jnp.int32),
            pltpu.VMEM_SHARED((NP, DH), jnp.float32),
            pltpu.SemaphoreType.DMA,
            pltpu.SemaphoreType.DMA,
            pltpu.SemaphoreType.DMA,
            pltpu.SemaphoreType.DMA,
            pltpu.SemaphoreType.DMA,
            pltpu.SemaphoreType.DMA,
        ],
    )
    def sc_k(tab_hbm, w_hbm, gi_hbm, dst_hbm, z_hbm, out_hbm,
             wv0, wv1, gr0, gr1, dv0, dv1, gia0, gia1,
             rows0, rows1, msg0, msg1, dsc0, dsc1, agg_sh,
             se0, se1, sg0, sg1, ssc0, ssc1):
        cid = lax.axis_index("c")
        sid = lax.axis_index("s")
        cbase = (sid if feat_split else cid * 16 + sid) * nchunks
        wv = (wv0, wv1)
        gr = (gr0, gr1)
        dv = (dv0, dv1)
        gia = (gia0, gia1)
        rows = (rows0, rows1)
        msg = (msg0, msg1)
        dsc = (dsc0, dsc1)
        se = (se0, se1)
        sg = (sg0, sg1)
        ssc = (ssc0, ssc1)

        pltpu.sync_copy(z_hbm.at[pl.ds(sid * nsl, nsl)],
                        agg_sh.at[pl.ds(sid * nsl, nsl)])
        plsc.subcore_barrier()

        def issue_edata(c, p):
            g = cbase + c
            pltpu.async_copy(w_hbm.at[pl.ds(g * (C8 * 16), C8 * 16)],
                             wv[p], se[p])
            pltpu.async_copy(gi_hbm.at[pl.ds(g * C8, C8)], gr[p], se[p])
            pltpu.async_copy(dst_hbm.at[pl.ds(g * C, C)], dv[p], se[p])

        def wait_edata(p):
            pltpu.make_async_copy(w_hbm.at[pl.ds(0, C8 * 16)], wv[p],
                                  se[p]).wait()
            pltpu.make_async_copy(gi_hbm.at[pl.ds(0, C8)], gr[p],
                                  se[p]).wait()
            pltpu.make_async_copy(dst_hbm.at[pl.ds(0, C)], dv[p],
                                  se[p]).wait()

        def build_gi_issue_gather(p):
            for j in range(C8 // 16):
                v = gr[p][pl.ds(j * 16, 16)]
                if feat_split:
                    v = v + cid * 28
                gia[p][pl.ds(j * 16, 16)] = v
            pltpu.async_copy(tab_hbm.at[gia[p]], rows[p], sg[p])

        def wait_gather(p):
            pltpu.make_async_copy(tab_hbm.at[gia[p]], rows[p], sg[p]).wait()

        def wait_scatter(p):
            pltpu.make_async_copy(msg[p], agg_sh.at[dsc[p]], ssc[p]).wait()

        def body(c, p, static_tail):
            q = 1 - p
            wait_gather(p)

            def prep_next():
                wait_edata(q)
                build_gi_issue_gather(q)

            if not static_tail:
                pl.when(c + 1 < nchunks)(prep_next)

            pl.when(c >= 2)(lambda: wait_scatter(p))

            def ebody(e, tok):
                accs = [jnp.zeros((16,), jnp.float32) for _ in range(DH // 16)]
                for s in range(8):
                    ws = wv[p][pl.ds((e * 8 + s) * 16, 16)]
                    for g in range(DH // 16):
                        accs[g] = accs[g] + ws * rows[p][e * 8 + s,
                                                         pl.ds(g * 16, 16)]
                for g in range(DH // 16):
                    msg[p][e, pl.ds(g * 16, 16)] = accs[g]
                return tok

            lax.fori_loop(0, C, ebody, jnp.int32(0))
            dsc[p][pl.ds(0, 16)] = dv[p][pl.ds(0, 16)]
            pltpu.async_copy(msg[p], agg_sh.at[dsc[p]], ssc[p], add=True)

            def prefetch():
                issue_edata(c + 2, p)

            if not static_tail:
                pl.when(c + 2 < nchunks)(prefetch)

        issue_edata(0, 0)
        issue_edata(1, 1)
        wait_edata(0)
        build_gi_issue_gather(0)

        def pair(i, tok):
            body(2 * i, 0, False)
            body(2 * i + 1, 1, False)
            return tok

        lax.fori_loop(0, nchunks // 2, pair, jnp.int32(0))
        if nchunks % 2:
            body(nchunks - 1, 0, True)
        wait_scatter(0)
        wait_scatter(1)

        plsc.subcore_barrier()
        pltpu.sync_copy(agg_sh.at[pl.ds(sid * nsl, nsl)],
                        out_hbm.at[pl.ds(cid * NP + sid * nsl, nsl)])

    return sc_k


def _place(vec, dout, big):
    if big:
        hl = dout // 2
        return jnp.concatenate([
            jnp.pad(vec[:hl], (0, DH - hl)),
            jnp.pad(vec[hl:], (0, DH - (dout - hl)))])[None]
    return jnp.pad(vec, (0, DH - dout))[None]


def kernel(x, edge_index, edge_attr, params):
    perm = jnp.argsort(edge_index[0])
    src = edge_index[0][perm]
    dst = edge_index[1][perm]
    edge_attr = edge_attr[perm]
    src_p = jnp.zeros((1, EA), jnp.int32).at[0, :N_EDGES].set(src)
    ea_p = jnp.zeros((EA, 3), jnp.float32).at[:N_EDGES].set(edge_attr)
    dst_p = jnp.zeros((EA,), jnp.int32).at[:N_EDGES].set(dst)

    g56, g28, wb = _basis(ea_p.T, src_p)
    wb = wb.at[:, N_EDGES:].set(0.0)
    w16 = jnp.broadcast_to(wb.T.reshape(EA * 8, 1), (EA * 8, 16)).reshape(-1)
    gi56 = g56.T.reshape(-1)
    gi28 = g28.T.reshape(-1)
    zeros_np = jnp.zeros((NP, DH), jnp.float32)

    h = jnp.zeros((NP, DH), jnp.float32).at[:N_NODES, :50].set(x)
    pin = np.arange(50)
    for i, (din, dout) in enumerate(LAYER_DIMS):
        big = dout > DH
        ns = 56 if big else 28
        din_p = h.shape[1]

        w28 = jnp.concatenate([params['W%d' % i],
                               params['root%d' % i][None]], axis=0)
        tmp = jnp.zeros((din_p, 28, dout), jnp.float32
                        ).at[pin, :, :].set(w28.transpose(1, 0, 2))
        if big:
            hl = dout // 2
            wext = jnp.concatenate([
                jnp.pad(tmp[:, :, :hl], ((0, 0), (0, 0), (0, DH - hl))),
                jnp.pad(tmp[:, :, hl:], ((0, 0), (0, 0), (0, DH - (dout - hl)))),
            ], axis=1).reshape(din_p, ns * DH)
        else:
            wext = jnp.pad(tmp, ((0, 0), (0, 0), (0, DH - dout))
                           ).reshape(din_p, ns * DH)

        bias_p = _place(params['bias%d' % i], dout, big)
        if i < 11:
            g_p = _place(params['gamma%d' % i] / jnp.sqrt(1.0 + 1e-5), dout, big)
            b_p = _place(params['beta%d' % i], dout, big)
        else:
            g_p = jnp.zeros_like(bias_p)
            b_p = jnp.zeros_like(bias_p)

        xwr, r = _make_mm(din_p, ns)(h, wext)
        agg = _make_sc(big)(xwr.reshape(NP * ns, DH),
                            w16, gi56 if big else gi28, dst_p, zeros_np)
        if big:
            h = _make_post_big(i == 11)(agg, r, bias_p, g_p, b_p)
            hl = dout // 2
            pin = np.concatenate([np.arange(hl), DH + np.arange(dout - hl)])
        else:
            h = _make_post_small(i == 11)(agg, agg, r, bias_p, g_p, b_p)
            pin = np.arange(dout)
    return h[:N_NODES, :50]


# speedup vs baseline: 1.1936x; 1.1936x over previous
import functools

import numpy as np
import jax
import jax.numpy as jnp
from jax import lax
from jax.experimental import pallas as pl
from jax.experimental.pallas import tpu as pltpu
from jax.experimental.pallas import tpu_sc as plsc

N_NODES = 10000
N_EDGES = 160000
NP = 10240
DH = 128
EA = 160768
C = 16
C8 = C * 8
EW = C * 8 * 16 + C * 8 + C
LAYER_DIMS = [(50, 75), (75, 100), (100, 200), (200, 200), (200, 200),
              (200, 200), (200, 200), (200, 200), (200, 200), (200, 100),
              (100, 75), (75, 50)]


def _basis_kernel(ea_ref, src_ref, g56_ref, g28_ref, wb_ref):
    p = jnp.clip(ea_ref[...], 0.0, 1.0) * 2.0
    lo = jnp.minimum(jnp.floor(p), 1.0)
    frac = p - lo
    lo_i = lo.astype(jnp.int32)
    strides = (1, 3, 9)
    for s in range(8):
        b = jnp.ones((1, EA), jnp.float32)
        wi = jnp.zeros((1, EA), jnp.int32)
        for dd in range(3):
            bit = (s >> dd) & 1
            f = frac[dd:dd + 1, :]
            b = b * (f if bit else (1.0 - f))
            wi = wi + (lo_i[dd:dd + 1, :] + bit) * strides[dd]
        g56_ref[s:s + 1, :] = src_ref[...] * 56 + wi
        g28_ref[s:s + 1, :] = src_ref[...] * 28 + wi
        wb_ref[s:s + 1, :] = b


_basis = pl.pallas_call(
    _basis_kernel,
    out_shape=(jax.ShapeDtypeStruct((8, EA), jnp.int32),
               jax.ShapeDtypeStruct((8, EA), jnp.int32),
               jax.ShapeDtypeStruct((8, EA), jnp.float32)),
)


@functools.cache
def _make_mm(din_p, ns):
    cols = ns * DH
    bm = 320

    def mm_body(h_ref, w_ref, o_ref, r_ref):
        acc = jnp.dot(h_ref[...], w_ref[...], preferred_element_type=jnp.float32)
        o_ref[...] = acc
        if ns == 56:
            r_ref[...] = jnp.concatenate(
                [acc[:, 27 * DH:28 * DH], acc[:, 55 * DH:56 * DH]], axis=1)
        else:
            r_ref[...] = acc[:, 27 * DH:28 * DH]

    rw = 2 * DH if ns == 56 else DH
    return pl.pallas_call(
        mm_body,
        grid=(NP // bm,),
        in_specs=[pl.BlockSpec((bm, din_p), lambda i: (i, 0)),
                  pl.BlockSpec((din_p, cols), lambda i: (0, 0))],
        out_specs=(pl.BlockSpec((bm, cols), lambda i: (i, 0)),
                   pl.BlockSpec((bm, rw), lambda i: (i, 0))),
        out_shape=(jax.ShapeDtypeStruct((NP, cols), jnp.float32),
                   jax.ShapeDtypeStruct((NP, rw), jnp.float32)),
    )


@functools.cache
def _make_post_big(final):
    bp = 640
    gi = NP // bp

    def post_body(agg_ref, r_ref, bias_ref, g_ref, b_ref, o_ref):
        v = agg_ref[...] + r_ref[...] + bias_ref[...]
        if not final:
            v = jnp.where(v > 0, v, jnp.exp(v) - 1.0)
            v = v * g_ref[...] + b_ref[...]
        o_ref[...] = v

    return pl.pallas_call(
        post_body,
        grid=(gi, 2),
        in_specs=[pl.BlockSpec((bp, DH), lambda i, j: (j * gi + i, 0)),
                  pl.BlockSpec((bp, DH), lambda i, j: (i, j)),
                  pl.BlockSpec((1, DH), lambda i, j: (0, j)),
                  pl.BlockSpec((1, DH), lambda i, j: (0, j)),
                  pl.BlockSpec((1, DH), lambda i, j: (0, j))],
        out_specs=pl.BlockSpec((bp, DH), lambda i, j: (i, j)),
        out_shape=jax.ShapeDtypeStruct((NP, 2 * DH), jnp.float32),
    )


@functools.cache
def _make_post_small(final):
    bp = 640
    gi = NP // bp

    def post_body(a0_ref, a1_ref, r_ref, bias_ref, g_ref, b_ref, o_ref):
        v = a0_ref[...] + a1_ref[...] + r_ref[...] + bias_ref[...]
        if not final:
            v = jnp.where(v > 0, v, jnp.exp(v) - 1.0)
            v = v * g_ref[...] + b_ref[...]
        o_ref[...] = v

    return pl.pallas_call(
        post_body,
        grid=(gi,),
        in_specs=[pl.BlockSpec((bp, DH), lambda i: (i, 0)),
                  pl.BlockSpec((bp, DH), lambda i: (gi + i, 0)),
                  pl.BlockSpec((bp, DH), lambda i: (i, 0)),
                  pl.BlockSpec((1, DH), lambda i: (0, 0)),
                  pl.BlockSpec((1, DH), lambda i: (0, 0)),
                  pl.BlockSpec((1, DH), lambda i: (0, 0))],
        out_specs=pl.BlockSpec((bp, DH), lambda i: (i, 0)),
        out_shape=jax.ShapeDtypeStruct((NP, DH), jnp.float32),
    )


@functools.cache
def _make_sc(feat_split):
    ept = EA // 16 if feat_split else EA // 32
    nchunks = ept // C
    nsl = NP // 16
    mesh = plsc.VectorSubcoreMesh(core_axis_name="c", subcore_axis_name="s")

    @functools.partial(
        pl.kernel,
        out_type=jax.ShapeDtypeStruct((2 * NP, DH), jnp.float32),
        mesh=mesh,
        scratch_types=[
            pltpu.VMEM((EW,), jnp.float32),
            pltpu.VMEM((EW,), jnp.float32),
            pltpu.VMEM((C8,), jnp.int32),
            pltpu.VMEM((C8,), jnp.int32),
            pltpu.VMEM((C8, DH), jnp.float32),
            pltpu.VMEM((C8, DH), jnp.float32),
            pltpu.VMEM((C, DH), jnp.float32),
            pltpu.VMEM((C, DH), jnp.float32),
            pltpu.VMEM((C,), jnp.int32),
            pltpu.VMEM((C,), jnp.int32),
            pltpu.VMEM_SHARED((NP, DH), jnp.float32),
            pltpu.SemaphoreType.DMA,
            pltpu.SemaphoreType.DMA,
            pltpu.SemaphoreType.DMA,
            pltpu.SemaphoreType.DMA,
            pltpu.SemaphoreType.DMA,
            pltpu.SemaphoreType.DMA,
        ],
    )
    def sc_k(tab_hbm, ed_hbm, z_hbm, out_hbm,
             ed0, ed1, gia0, gia1,
             rows0, rows1, msg0, msg1, dsc0, dsc1, agg_sh,
             se0, se1, sg0, sg1, ssc0, ssc1):
        cid = lax.axis_index("c")
        sid = lax.axis_index("s")
        cbase = (sid if feat_split else cid * 16 + sid) * nchunks
        ed = (ed0, ed1)
        gia = (gia0, gia1)
        rows = (rows0, rows1)
        msg = (msg0, msg1)
        dsc = (dsc0, dsc1)
        se = (se0, se1)
        sg = (sg0, sg1)
        ssc = (ssc0, ssc1)

        pltpu.sync_copy(z_hbm.at[pl.ds(sid * nsl, nsl)],
                        agg_sh.at[pl.ds(sid * nsl, nsl)])
        plsc.subcore_barrier()

        def issue_edata(c, p):
            g = cbase + c
            pltpu.async_copy(ed_hbm.at[pl.ds(g * EW, EW)], ed[p], se[p])

        def wait_edata(p):
            pltpu.make_async_copy(ed_hbm.at[pl.ds(0, EW)], ed[p],
                                  se[p]).wait()

        def build_gi_issue_gather(p):
            off = C8 * 16
            for j in range(C8 // 16):
                v = ed[p][pl.ds(off + j * 16, 16)].astype(jnp.int32)
                if feat_split:
                    v = v + cid * 28
                gia[p][pl.ds(j * 16, 16)] = v
            pltpu.async_copy(tab_hbm.at[gia[p]], rows[p], sg[p])

        def wait_gather(p):
            pltpu.make_async_copy(tab_hbm.at[gia[p]], rows[p], sg[p]).wait()

        def wait_scatter(p):
            pltpu.make_async_copy(msg[p], agg_sh.at[dsc[p]], ssc[p]).wait()

        def body(c, p, static_tail):
            q = 1 - p
            wait_gather(p)

            def prep_next():
                wait_edata(q)
                build_gi_issue_gather(q)

            if not static_tail:
                pl.when(c + 1 < nchunks)(prep_next)

            pl.when(c >= 2)(lambda: wait_scatter(p))

            def ebody(e2, tok):
                for ee in range(2):
                    e = e2 * 2 + ee
                    accs = [jnp.zeros((16,), jnp.float32)
                            for _ in range(DH // 16)]
                    for s in range(8):
                        ws = ed[p][pl.ds((e * 8 + s) * 16, 16)]
                        for g in range(DH // 16):
                            accs[g] = accs[g] + ws * rows[p][e * 8 + s,
                                                             pl.ds(g * 16, 16)]
                    for g in range(DH // 16):
                        msg[p][e, pl.ds(g * 16, 16)] = accs[g]
                return tok

            lax.fori_loop(0, C // 2, ebody, jnp.int32(0))
            dsc[p][pl.ds(0, 16)] = ed[p][pl.ds(C8 * 16 + C8, 16)
                                         ].astype(jnp.int32)
            pltpu.async_copy(msg[p], agg_sh.at[dsc[p]], ssc[p], add=True)

            def prefetch():
                issue_edata(c + 2, p)

            if not static_tail:
                pl.when(c + 2 < nchunks)(prefetch)

        issue_edata(0, 0)
        issue_edata(1, 1)
        wait_edata(0)
        build_gi_issue_gather(0)

        def pair(i, tok):
            body(2 * i, 0, False)
            body(2 * i + 1, 1, False)
            return tok

        lax.fori_loop(0, nchunks // 2, pair, jnp.int32(0))
        if nchunks % 2:
            body(nchunks - 1, 0, True)
        wait_scatter(0)
        wait_scatter(1)

        plsc.subcore_barrier()
        pltpu.sync_copy(agg_sh.at[pl.ds(sid * nsl, nsl)],
                        out_hbm.at[pl.ds(cid * NP + sid * nsl, nsl)])

    return sc_k


def _place(vec, dout, big):
    if big:
        hl = dout // 2
        return jnp.concatenate([
            jnp.pad(vec[:hl], (0, DH - hl)),
            jnp.pad(vec[hl:], (0, DH - (dout - hl)))])[None]
    return jnp.pad(vec, (0, DH - dout))[None]


def kernel(x, edge_index, edge_attr, params):
    src = edge_index[0]
    dst = edge_index[1]
    src_p = jnp.zeros((1, EA), jnp.int32).at[0, :N_EDGES].set(src)
    ea_p = jnp.zeros((EA, 3), jnp.float32).at[:N_EDGES].set(edge_attr)
    dst_p = jnp.zeros((EA,), jnp.int32).at[:N_EDGES].set(dst)

    g56, g28, wb = _basis(ea_p.T, src_p)
    wb = wb.at[:, N_EDGES:].set(0.0)
    nch = EA // C
    w16 = jnp.broadcast_to(wb.T.reshape(EA * 8, 1), (EA * 8, 16))

    def pack(gi_arr):
        return jnp.concatenate(
            [w16.reshape(nch, C8 * 16),
             gi_arr.T.reshape(nch, C8).astype(jnp.float32),
             dst_p.reshape(nch, C).astype(jnp.float32)], axis=1).reshape(-1)

    ed56 = pack(g56)
    ed28 = pack(g28)
    zeros_np = jnp.zeros((NP, DH), jnp.float32)

    h = jnp.zeros((NP, DH), jnp.float32).at[:N_NODES, :50].set(x)
    pin = np.arange(50)
    for i, (din, dout) in enumerate(LAYER_DIMS):
        big = dout > DH
        ns = 56 if big else 28
        din_p = h.shape[1]

        w28 = jnp.concatenate([params['W%d' % i],
                               params['root%d' % i][None]], axis=0)
        tmp = jnp.zeros((din_p, 28, dout), jnp.float32
                        ).at[pin, :, :].set(w28.transpose(1, 0, 2))
        if big:
            hl = dout // 2
            wext = jnp.concatenate([
                jnp.pad(tmp[:, :, :hl], ((0, 0), (0, 0), (0, DH - hl))),
                jnp.pad(tmp[:, :, hl:], ((0, 0), (0, 0), (0, DH - (dout - hl)))),
            ], axis=1).reshape(din_p, ns * DH)
        else:
            wext = jnp.pad(tmp, ((0, 0), (0, 0), (0, DH - dout))
                           ).reshape(din_p, ns * DH)

        bias_p = _place(params['bias%d' % i], dout, big)
        if i < 11:
            g_p = _place(params['gamma%d' % i] / jnp.sqrt(1.0 + 1e-5), dout, big)
            b_p = _place(params['beta%d' % i], dout, big)
        else:
            g_p = jnp.zeros_like(bias_p)
            b_p = jnp.zeros_like(bias_p)

        xwr, r = _make_mm(din_p, ns)(h, wext)
        agg = _make_sc(big)(xwr.reshape(NP * ns, DH),
                            ed56 if big else ed28, zeros_np)
        if big:
            h = _make_post_big(i == 11)(agg, r, bias_p, g_p, b_p)
            hl = dout // 2
            pin = np.concatenate([np.arange(hl), DH + np.arange(dout - hl)])
        else:
            h = _make_post_small(i == 11)(agg, agg, r, bias_p, g_p, b_p)
            pin = np.arange(dout)
    return h[:N_NODES, :50]
